# Initial kernel scaffold; baseline (speedup 1.0000x reference)
import jax
import jax.numpy as jnp
from jax.experimental import pallas as pl


def kernel(x, V, E, enc_ln0_g, enc_ln0_b, enc_W1, enc_b1, enc_ln1_g, enc_ln1_b, enc_W2, enc_b2, dec_ln0_g, dec_ln0_b, dec_W1, dec_b1, dec_ln1_g, dec_ln1_b, dec_W2, dec_b2):
    raise NotImplementedError("write your pallas kernel here")


# speedup vs baseline: 3.5136x; 3.5136x over previous
import functools

import jax
import jax.numpy as jnp
from jax import lax
from jax.experimental import pallas as pl
from jax.experimental.pallas import tpu as pltpu
from jax.experimental.pallas import tpu_sc as plsc

N = 10000
NNZ = 320000
D = 128

NC = 2
NS = 16
NW = NC * NS
CHUNK = 128
PER_W = 10240
CPW = PER_W // CHUNK
PAD_TOTAL = NW * PER_W
NPAD = 10112
ROWS_PER_TILE = NPAD // NS
DUMMY_ROW = NPAD - 1

_EPS = 1e-5


def _ln_mlp(xb, g0, b0, w1t, b1, g1, b1_, w2t, b2):
    m = jnp.mean(xb, axis=-1, keepdims=True)
    v = jnp.mean((xb - m) ** 2, axis=-1, keepdims=True)
    xn = (xb - m) * lax.rsqrt(v + _EPS) * g0 + b0
    hh = jnp.maximum(jnp.dot(xn, w1t, preferred_element_type=jnp.float32) + b1, 0.0)
    m2 = jnp.mean(hh, axis=-1, keepdims=True)
    v2 = jnp.mean((hh - m2) ** 2, axis=-1, keepdims=True)
    hn = (hh - m2) * lax.rsqrt(v2 + _EPS) * g1 + b1_
    return jnp.dot(hn, w2t, preferred_element_type=jnp.float32) + b2


def _enc_body(x_ref, g0, b0, w1t, b1, g1, b1_, w2t, b2, o_ref):
    o_ref[...] = jnp.maximum(
        _ln_mlp(x_ref[...], g0[...], b0[...], w1t[...], b1[...], g1[...],
                b1_[...], w2t[...], b2[...]), 0.0)


def _dec_body(p0_ref, p1_ref, g0, b0, w1t, b1, g1, b1_, w2t, b2, o_ref):
    agg = p0_ref[0] + p1_ref[0]
    o_ref[...] = jnp.maximum(
        _ln_mlp(agg, g0[...], b0[...], w1t[...], b1[...], g1[...],
                b1_[...], w2t[...], b2[...]), 0.0)


_ROWS_BLK = 1000
_GRID = N // _ROWS_BLK

_W_SPECS = [pl.BlockSpec((1, D), lambda i: (0, 0)) for _ in range(4)]


def _wspecs():
    vec = lambda i: (0, 0)
    return [
        pl.BlockSpec((1, D), vec),
        pl.BlockSpec((1, D), vec),
        pl.BlockSpec((D, D), vec),
        pl.BlockSpec((1, D), vec),
        pl.BlockSpec((1, D), vec),
        pl.BlockSpec((1, D), vec),
        pl.BlockSpec((D, D), vec),
        pl.BlockSpec((1, D), vec),
    ]


def _enc_call(x, *w):
    return pl.pallas_call(
        _enc_body,
        grid=(_GRID,),
        in_specs=[pl.BlockSpec((_ROWS_BLK, D), lambda i: (i, 0))] + _wspecs(),
        out_specs=pl.BlockSpec((_ROWS_BLK, D), lambda i: (i, 0)),
        out_shape=jax.ShapeDtypeStruct((N, D), jnp.float32),
    )(x, *w)


def _dec_call(parts, *w):
    return pl.pallas_call(
        _dec_body,
        grid=(_GRID,),
        in_specs=[
            pl.BlockSpec((1, _ROWS_BLK, D), lambda i: (0, i, 0)),
            pl.BlockSpec((1, _ROWS_BLK, D), lambda i: (1, i, 0)),
        ] + _wspecs(),
        out_specs=pl.BlockSpec((_ROWS_BLK, D), lambda i: (i, 0)),
        out_shape=jax.ShapeDtypeStruct((N, D), jnp.float32),
    )(parts, parts, *w)


@functools.cache
def _sc_call():
    mesh = plsc.VectorSubcoreMesh(
        core_axis_name="c", subcore_axis_name="s",
        num_cores=NC, num_subcores=NS)
    return pl.kernel(
        _gather_segsum,
        out_type=jax.ShapeDtypeStruct((NC, NPAD, D), jnp.float32),
        mesh=mesh,
        scratch_types=[
            pltpu.VMEM((CPW, CHUNK), jnp.int32),
            pltpu.VMEM((CPW, CHUNK), jnp.int32),
            pltpu.VMEM((CHUNK, D), jnp.float32),
            pltpu.VMEM_SHARED((NPAD, D), jnp.float32),
            pltpu.SemaphoreType.DMA,
        ],
    )


def _gather_segsum(h_hbm, v_hbm, e_hbm, z_hbm, out_hbm,
                   idxv, idxe, rows, acc, sem):
    c = lax.axis_index("c")
    s = lax.axis_index("s")
    wid = s * NC + c
    r0 = s * ROWS_PER_TILE

    pltpu.sync_copy(z_hbm.at[pl.ds(r0, ROWS_PER_TILE)],
                    acc.at[pl.ds(r0, ROWS_PER_TILE)])
    pltpu.sync_copy(v_hbm.at[pl.ds(wid * CPW, CPW)], idxv)
    pltpu.sync_copy(e_hbm.at[pl.ds(wid * CPW, CPW)], idxe)
    plsc.subcore_barrier()

    def chunk(j, carry):
        pltpu.async_copy(h_hbm.at[idxv.at[j]], rows, sem).wait()
        pltpu.sync_copy(rows, acc.at[idxe.at[j]], add=True)
        return carry

    lax.fori_loop(0, CPW, chunk, 0)
    plsc.subcore_barrier()

    pltpu.sync_copy(acc.at[pl.ds(r0, ROWS_PER_TILE)],
                    out_hbm.at[c, pl.ds(r0, ROWS_PER_TILE)])


def kernel(x, V, E, enc_ln0_g, enc_ln0_b, enc_W1, enc_b1, enc_ln1_g,
           enc_ln1_b, enc_W2, enc_b2, dec_ln0_g, dec_ln0_b, dec_W1, dec_b1,
           dec_ln1_g, dec_ln1_b, dec_W2, dec_b2):
    pad = PAD_TOTAL - NNZ
    Vp = jnp.concatenate([V, jnp.zeros((pad,), jnp.int32)]).reshape(NW * CPW, CHUNK)
    Ep = jnp.concatenate(
        [E, jnp.full((pad,), DUMMY_ROW, jnp.int32)]).reshape(NW * CPW, CHUNK)
    zeros = jnp.zeros((NPAD, D), jnp.float32)

    r = lambda a: a.reshape(1, D)
    enc_w = (r(enc_ln0_g), r(enc_ln0_b), enc_W1.T, r(enc_b1),
             r(enc_ln1_g), r(enc_ln1_b), enc_W2.T, r(enc_b2))
    dec_w = (r(dec_ln0_g), r(dec_ln0_b), dec_W1.T, r(dec_b1),
             r(dec_ln1_g), r(dec_ln1_b), dec_W2.T, r(dec_b2))

    h = _enc_call(x, *enc_w)
    parts = _sc_call()(h, Vp, Ep, zeros)
    return _dec_call(parts, *dec_w)
